# baseline (speedup 1.0000x reference)
import functools

import jax
import jax.numpy as jnp
from jax import lax
from jax.experimental import pallas as pl
from jax.experimental.pallas import tpu as pltpu
from jax.experimental.pallas import tpu_sc as plsc

NC = 2
NS = 16
L = 16
NW = NC * NS
ROWS_OUT = 1024
ROWS_IN = 512
D = 12 * 64
C = ROWS_OUT // NW
IDX_CHUNKS = ROWS_IN // L

_mesh = plsc.VectorSubcoreMesh(core_axis_name="c", subcore_axis_name="s")


@functools.partial(
    pl.kernel,
    out_type=jax.ShapeDtypeStruct((ROWS_OUT, D), jnp.float32),
    mesh=_mesh,
    scratch_types=[
        pltpu.VMEM((ROWS_IN,), jnp.int32),
        pltpu.VMEM((C, D), jnp.float32),
        pltpu.VMEM((C,), jnp.int32),
    ],
    compiler_params=pltpu.CompilerParams(use_tc_tiling_on_sc=False,
                                         needs_layout_passes=False),
)
def _index_put_sc(idx_hbm, kval_hbm, cache_hbm, out_hbm, idx_v, buf, slot_map):
    wid = lax.axis_index("s") * NC + lax.axis_index("c")
    base = wid * C

    pltpu.sync_copy(idx_hbm, idx_v)

    acc_lo = jnp.zeros((L,), jnp.int32)
    acc_in = jnp.zeros((L,), jnp.int32)
    one = jnp.ones((L,), jnp.int32)
    zero = jnp.zeros((L,), jnp.int32)
    for v in range(IDX_CHUNKS):
        vec = idx_v[pl.ds(v * L, L)]
        acc_lo = acc_lo + jnp.where(vec < base, one, zero)
        acc_in = acc_in + jnp.where((vec >= base) & (vec < base + C), one, zero)
    lo = jnp.sum(acc_lo)
    cnt = jnp.sum(acc_in)

    @pl.when(cnt == C)
    def _():
        pltpu.sync_copy(kval_hbm.at[pl.ds(lo, C)], out_hbm.at[pl.ds(base, C)])

    @pl.when(cnt == 0)
    def _():
        pltpu.sync_copy(cache_hbm.at[pl.ds(base, C)],
                        out_hbm.at[pl.ds(base, C)])

    @pl.when((cnt > 0) & (cnt < C))
    def _():
        pltpu.sync_copy(cache_hbm.at[pl.ds(base, C)], buf)
        neg = jnp.full((L,), -1, jnp.int32)
        for s in range(C // L):
            slot_map[pl.ds(s * L, L)] = neg
        for v in range(IDX_CHUNKS):
            vec = idx_v[pl.ds(v * L, L)]
            rel = vec - base
            m = (rel >= 0) & (rel < C)
            src = lax.iota(jnp.int32, L) + (v * L)
            plsc.store_scatter(slot_map, [jnp.where(m, rel, 0)], src, mask=m)
        lanes = lax.iota(jnp.int32, L)
        for j in range(C):
            vec = slot_map[pl.ds((j // L) * L, L)]
            row = jnp.sum(jnp.where(lanes == (j % L), vec, zero))

            @pl.when(row >= 0)
            def _():
                pltpu.sync_copy(kval_hbm.at[pl.ds(row, 1)],
                                buf.at[pl.ds(j, 1)])

        pltpu.sync_copy(buf, out_hbm.at[pl.ds(base, C)])


def kernel(input_pos, k_val, k_cache):
    idx = input_pos.astype(jnp.int32)
    kval2d = k_val.reshape(ROWS_IN, D)
    cache2d = k_cache.reshape(ROWS_OUT, D)
    out2d = _index_put_sc(idx, kval2d, cache2d)
    return out2d.reshape(k_cache.shape)


# speedup vs baseline: 3.6312x; 3.6312x over previous
import functools

import jax
import jax.numpy as jnp
from jax import lax
from jax.experimental import pallas as pl
from jax.experimental.pallas import tpu as pltpu
from jax.experimental.pallas import tpu_sc as plsc

NC = 2
NS = 16
L = 16
NW = NC * NS
ROWS_OUT = 1024
ROWS_IN = 512
D = 12 * 64
C = ROWS_OUT // NW
IDX_CHUNKS = ROWS_IN // L

_mesh = plsc.VectorSubcoreMesh(core_axis_name="c", subcore_axis_name="s")


@functools.partial(
    pl.kernel,
    out_type=jax.ShapeDtypeStruct((ROWS_OUT, D), jnp.float32),
    mesh=_mesh,
    scratch_types=[
        pltpu.VMEM((ROWS_IN,), jnp.int32),
        pltpu.VMEM((C, D), jnp.float32),
        pltpu.VMEM((C,), jnp.int32),
    ],
    compiler_params=pltpu.CompilerParams(use_tc_tiling_on_sc=False,
                                         needs_layout_passes=False),
)
def _index_put_sc(idx_hbm, kval_hbm, cache_hbm, out_hbm, idx_v, buf, slot_map):
    wid = lax.axis_index("s") * NC + lax.axis_index("c")
    base = wid * C

    pltpu.sync_copy(idx_hbm, idx_v)

    acc_lo = jnp.zeros((L,), jnp.int32)
    acc_in = jnp.zeros((L,), jnp.int32)
    one = jnp.ones((L,), jnp.int32)
    zero = jnp.zeros((L,), jnp.int32)
    for v in range(IDX_CHUNKS):
        vec = idx_v[pl.ds(v * L, L)]
        acc_lo = acc_lo + jnp.where(vec < base, one, zero)
        acc_in = acc_in + jnp.where((vec >= base) & (vec < base + C), one, zero)
    lo = jnp.sum(acc_lo)
    cnt = jnp.sum(acc_in)

    @pl.when(cnt == C)
    def _():
        pltpu.sync_copy(kval_hbm.at[pl.ds(lo, C)], buf)
        pltpu.sync_copy(buf, out_hbm.at[pl.ds(base, C)])

    @pl.when(cnt == 0)
    def _():
        pltpu.sync_copy(cache_hbm.at[pl.ds(base, C)], buf)
        pltpu.sync_copy(buf, out_hbm.at[pl.ds(base, C)])

    @pl.when((cnt > 0) & (cnt < C))
    def _():
        pltpu.sync_copy(cache_hbm.at[pl.ds(base, C)], buf)
        neg = jnp.full((L,), -1, jnp.int32)
        for s in range(C // L):
            slot_map[pl.ds(s * L, L)] = neg
        for v in range(IDX_CHUNKS):
            vec = idx_v[pl.ds(v * L, L)]
            rel = vec - base
            m = (rel >= 0) & (rel < C)
            src = lax.iota(jnp.int32, L) + (v * L)
            plsc.store_scatter(slot_map, [jnp.where(m, rel, 0)], src, mask=m)
        lanes = lax.iota(jnp.int32, L)
        for j in range(C):
            vec = slot_map[pl.ds((j // L) * L, L)]
            row = jnp.sum(jnp.where(lanes == (j % L), vec, zero))

            @pl.when(row >= 0)
            def _():
                pltpu.sync_copy(kval_hbm.at[pl.ds(row, 1)],
                                buf.at[pl.ds(j, 1)])

        pltpu.sync_copy(buf, out_hbm.at[pl.ds(base, C)])


def kernel(input_pos, k_val, k_cache):
    idx = input_pos.astype(jnp.int32)
    kval2d = k_val.reshape(ROWS_IN, D)
    cache2d = k_cache.reshape(ROWS_OUT, D)
    out2d = _index_put_sc(idx, kval2d, cache2d)
    return out2d.reshape(k_cache.shape)
